# baseline (speedup 1.0000x reference)
import functools

import jax
import jax.numpy as jnp
from jax import lax
from jax.experimental import pallas as pl
from jax.experimental.pallas import tpu as pltpu
from jax.experimental.pallas import tpu_sc as plsc

_B, _T = 4096, 50
_NTOT = _B * _T
_DIMS = (64, 64, 32, 32)
_OFFS = (0, 64, 128, 160)
_DSUM = 192
_NC, _NS = 2, 16
_NW = _NC * _NS
_BPW = _NTOT // _NW
_CHUNK = 128
_NCH = _BPW // _CHUNK

_mesh = plsc.VectorSubcoreMesh(core_axis_name="c", subcore_axis_name="s")


@functools.partial(
    pl.kernel,
    out_type=jax.ShapeDtypeStruct((_NTOT, _DSUM), jnp.float32),
    mesh=_mesh,
    compiler_params=pltpu.CompilerParams(use_tc_tiling_on_sc=False),
    scratch_types=[
        pltpu.VMEM((_BPW,), jnp.int32),
        pltpu.VMEM((_BPW,), jnp.int32),
        pltpu.VMEM((_BPW,), jnp.int32),
        pltpu.VMEM((_BPW,), jnp.int32),
        pltpu.VMEM((_CHUNK, 128), jnp.float32),
        pltpu.VMEM((_CHUNK, 128), jnp.float32),
        pltpu.VMEM((_CHUNK, 128), jnp.float32),
        pltpu.VMEM((_CHUNK, 128), jnp.float32),
        pltpu.SemaphoreType.DMA,
        pltpu.SemaphoreType.DMA,
    ],
)
def _emb_gather(x0_h, x1_h, x2_h, x3_h, w0_h, w1_h, w2_h, w3_h, out_h,
                i0, i1, i2, i3, s0, s1, s2, s3, gsem, osem):
    wid = lax.axis_index("s") * _NC + lax.axis_index("c")
    base = wid * _BPW

    pltpu.sync_copy(x0_h.at[pl.ds(base, _BPW)], i0)
    pltpu.sync_copy(x1_h.at[pl.ds(base, _BPW)], i1)
    pltpu.sync_copy(x2_h.at[pl.ds(base, _BPW)], i2)
    pltpu.sync_copy(x3_h.at[pl.ds(base, _BPW)], i3)

    idx_refs = (i0, i1, i2, i3)
    w_refs = (w0_h, w1_h, w2_h, w3_h)
    stages = (s0, s1, s2, s3)

    def chunk_body(j):
        off = pl.multiple_of(j * _CHUNK, _CHUNK)
        handles = []
        for k in range(4):
            src = w_refs[k].at[idx_refs[k].at[pl.ds(off, _CHUNK)]]
            handles.append(pltpu.async_copy(src, stages[k], gsem))
        for h in handles:
            h.wait()
        ohandles = []
        for k in range(4):
            dst = out_h.at[pl.ds(base + off, _CHUNK), pl.ds(_OFFS[k], _DIMS[k])]
            src = stages[k].at[:, pl.ds(0, _DIMS[k])]
            ohandles.append(pltpu.async_copy(src, dst, osem))
        for h in ohandles:
            h.wait()

    pl.loop(0, _NCH)(chunk_body)


def kernel(x0, x1, x2, x3, W0, W1, W2, W3):
    xs = [x.reshape(-1).astype(jnp.int32) for x in (x0, x1, x2, x3)]
    ws = [jnp.pad(w, ((0, 0), (0, 128 - w.shape[1]))) for w in (W0, W1, W2, W3)]
    out = _emb_gather(xs[0], xs[1], xs[2], xs[3], ws[0], ws[1], ws[2], ws[3])
    return out.reshape(_B, _T, _DSUM)


# speedup vs baseline: 1.2640x; 1.2640x over previous
import functools

import jax
import jax.numpy as jnp
from jax import lax
from jax.experimental import pallas as pl
from jax.experimental.pallas import tpu as pltpu
from jax.experimental.pallas import tpu_sc as plsc

_B, _T = 4096, 50
_NTOT = _B * _T
_DIMS = (64, 64, 32, 32)
_OFFS = (0, 64, 128, 160)
_DSUM = 192
_NC, _NS = 2, 16
_NW = _NC * _NS
_BPW = _B // _NW
_LPW = _BPW * _T
_NBC = 4
_CHUNK = _NBC * _T
_NCH = _BPW // _NBC
_GSPLIT = ((0, 128), (128, _CHUNK - 128))

_mesh = plsc.VectorSubcoreMesh(core_axis_name="c", subcore_axis_name="s")


@functools.partial(
    pl.kernel,
    out_type=jax.ShapeDtypeStruct((_B, _T, _DSUM), jnp.float32),
    mesh=_mesh,
    compiler_params=pltpu.CompilerParams(use_tc_tiling_on_sc=False),
    scratch_types=[
        pltpu.VMEM((_LPW,), jnp.int32),
        pltpu.VMEM((_LPW,), jnp.int32),
        pltpu.VMEM((_LPW,), jnp.int32),
        pltpu.VMEM((_LPW,), jnp.int32),
        pltpu.VMEM((2, _CHUNK, 64), jnp.float32),
        pltpu.VMEM((2, _CHUNK, 64), jnp.float32),
        pltpu.VMEM((2, _CHUNK, 32), jnp.float32),
        pltpu.VMEM((2, _CHUNK, 32), jnp.float32),
        pltpu.SemaphoreType.DMA,
        pltpu.SemaphoreType.DMA,
        pltpu.SemaphoreType.DMA,
        pltpu.SemaphoreType.DMA,
    ],
)
def _emb_gather(x0_h, x1_h, x2_h, x3_h, w0_h, w1_h, w2_h, w3_h, out_h,
                i0, i1, i2, i3, s0, s1, s2, s3, gsem0, gsem1, osem0, osem1):
    wid = lax.axis_index("s") * _NC + lax.axis_index("c")
    base = wid * _LPW
    bbase = wid * _BPW

    pltpu.sync_copy(x0_h.at[pl.ds(base, _LPW)], i0)
    pltpu.sync_copy(x1_h.at[pl.ds(base, _LPW)], i1)
    pltpu.sync_copy(x2_h.at[pl.ds(base, _LPW)], i2)
    pltpu.sync_copy(x3_h.at[pl.ds(base, _LPW)], i3)

    idx_refs = (i0, i1, i2, i3)
    w_refs = (w0_h, w1_h, w2_h, w3_h)
    stages = (s0, s1, s2, s3)
    gsems = (gsem0, gsem1)
    osems = (osem0, osem1)

    def gather_copies(j, sl):
        off = j * _CHUNK
        for k in range(4):
            for roff, glen in _GSPLIT:
                src = w_refs[k].at[idx_refs[k].at[pl.ds(off + roff, glen)]]
                dst = stages[k].at[sl, pl.ds(roff, glen)]
                yield src, dst, gsems[sl]

    def out_copies(j, sl):
        for bb in range(_NBC):
            b = bbase + j * _NBC + bb
            for k in range(4):
                src = stages[k].at[sl, pl.ds(bb * _T, _T)]
                dst = out_h.at[b, :, pl.ds(_OFFS[k], _DIMS[k])]
                yield src, dst, osems[sl]

    def fire(copies):
        for src, dst, sem in copies:
            pltpu.async_copy(src, dst, sem)

    def drain(copies):
        for src, dst, sem in copies:
            pltpu.make_async_copy(src, dst, sem).wait()

    fire(gather_copies(0, 0))

    def pair_body(j):
        for sl in range(2):
            cj = j + sl

            @pl.when(cj >= 1)
            def _():
                drain(out_copies(cj - 1, 1 - sl))

            @pl.when(cj + 1 <= _NCH - 1)
            def _():
                fire(gather_copies(cj + 1, 1 - sl))

            drain(gather_copies(cj, sl))
            fire(out_copies(cj, sl))

    pl.loop(0, _NCH, step=2)(pair_body)
    drain(out_copies(_NCH - 1, 1))


def kernel(x0, x1, x2, x3, W0, W1, W2, W3):
    xs = [x.reshape(-1).astype(jnp.int32) for x in (x0, x1, x2, x3)]
    return _emb_gather(xs[0], xs[1], xs[2], xs[3], W0, W1, W2, W3)


# speedup vs baseline: 1.3573x; 1.0738x over previous
import functools

import jax
import jax.numpy as jnp
from jax import lax
from jax.experimental import pallas as pl
from jax.experimental.pallas import tpu as pltpu
from jax.experimental.pallas import tpu_sc as plsc

_B, _T = 4096, 50
_TP = 64
_DIMS = (64, 64, 32, 32)
_OFFS = (0, 64, 128, 160)
_DSUM = 192
_NC, _NS = 2, 16
_NW = _NC * _NS
_BPW = _B // _NW
_L = 16

_mesh = plsc.VectorSubcoreMesh(core_axis_name="c", subcore_axis_name="s")


@functools.partial(
    pl.kernel,
    out_type=jax.ShapeDtypeStruct((_B, _T, _DSUM), jnp.float32),
    mesh=_mesh,
    scratch_types=[
        pltpu.VMEM((_BPW * _TP,), jnp.int32),
        pltpu.VMEM((_BPW * _TP,), jnp.int32),
        pltpu.VMEM((_BPW * _TP,), jnp.int32),
        pltpu.VMEM((_BPW * _TP,), jnp.int32),
        pltpu.VMEM((2, _T, _DSUM), jnp.float32),
        pltpu.VMEM((2, _T, 128), jnp.float32),
        pltpu.VMEM((2, _T, 128), jnp.float32),
        pltpu.VMEM((2, _T, 128), jnp.float32),
        pltpu.SemaphoreType.DMA,
        pltpu.SemaphoreType.DMA,
        pltpu.SemaphoreType.DMA,
        pltpu.SemaphoreType.DMA,
    ],
)
def _emb_gather(x0_h, x1_h, x2_h, x3_h, w0_h, w1_h, w2_h, w3_h, out_h,
                i0, i1, i2, i3, so, s1, s2, s3, gsem0, gsem1, osem0, osem1):
    wid = lax.axis_index("s") * _NC + lax.axis_index("c")
    base = wid * _BPW * _TP
    bbase = wid * _BPW

    pltpu.sync_copy(x0_h.at[pl.ds(base, _BPW * _TP)], i0)
    pltpu.sync_copy(x1_h.at[pl.ds(base, _BPW * _TP)], i1)
    pltpu.sync_copy(x2_h.at[pl.ds(base, _BPW * _TP)], i2)
    pltpu.sync_copy(x3_h.at[pl.ds(base, _BPW * _TP)], i3)

    idx_refs = (i0, i1, i2, i3)
    w_refs = (w0_h, w1_h, w2_h, w3_h)
    side = (s1, s2, s3)
    gsems = (gsem0, gsem1)
    osems = (osem0, osem1)

    def gather_copies(j, sl):
        off = j * _TP
        srcs = [w_refs[k].at[idx_refs[k].at[pl.ds(off, _T)]] for k in range(4)]
        yield srcs[0], so.at[sl, :, pl.ds(0, 128)], gsems[sl]
        for k in range(1, 4):
            yield srcs[k], side[k - 1].at[sl], gsems[sl]

    def out_copies(j, sl):
        yield so.at[sl], out_h.at[bbase + j], osems[sl]

    def fire(copies):
        for src, dst, sem in copies:
            pltpu.async_copy(src, dst, sem)

    def drain(copies):
        for src, dst, sem in copies:
            pltpu.make_async_copy(src, dst, sem).wait()

    def assemble(sl):
        for t in range(_T):
            for k in range(1, 4):
                sref = side[k - 1]
                for c in range(0, _DIMS[k], _L):
                    so[sl, t, pl.ds(_OFFS[k] + c, _L)] = sref[sl, t, pl.ds(c, _L)]

    fire(gather_copies(0, 0))

    def pair_body(j):
        for sl in range(2):
            cj = j + sl

            @pl.when(cj >= 1)
            def _():
                drain(out_copies(cj - 1, 1 - sl))

            @pl.when(cj + 1 <= _BPW - 1)
            def _():
                fire(gather_copies(cj + 1, 1 - sl))

            drain(gather_copies(cj, sl))
            assemble(sl)
            fire(out_copies(cj, sl))

    pl.loop(0, _BPW, step=2)(pair_body)
    drain(out_copies(_BPW - 1, 1))


def kernel(x0, x1, x2, x3, W0, W1, W2, W3):
    xs = [
        jnp.pad(x.astype(jnp.int32), ((0, 0), (0, _TP - _T))).reshape(-1)
        for x in (x0, x1, x2, x3)
    ]
    ws = [jnp.pad(w, ((0, 0), (0, 128 - w.shape[1]))) for w in (W0, W1, W2, W3)]
    return _emb_gather(xs[0], xs[1], xs[2], xs[3], ws[0], ws[1], ws[2], ws[3])
